# baseline (speedup 1.0000x reference)
import jax
import jax.numpy as jnp
from jax import lax
from jax.experimental import pallas as pl
from jax.experimental.pallas import tpu as pltpu
from jax.experimental.pallas import tpu_sc as plsc

_N = 10000
_E = 320000
_D = 128

_NC = 2
_NS = 16
_NW = _NC * _NS

_EPW = 10240
_EPAD = _EPW * _NW
_NPAD = _N + 8
_SLAB = 624
_TAIL = _NPAD - _SLAB * _NS
_DTAIL = _N - _SLAB * _NS

_CHD = 128
_NCHD = _EPW // _CHD

_CH = 64
_NCHA = _EPAD // _CH
_SG = 32
_PIPE = 4
_FAST = 0
_CF = 224
_CS = _NCHA // _NS - _CF

_mesh = plsc.VectorSubcoreMesh(core_axis_name="c", subcore_axis_name="s")


def _deg_body(dst_hbm, ones_hbm, zeros_hbm, out_hbm, didx, ones_v, acc,
              dsem0, dsem1):
    c = lax.axis_index("c")
    s = lax.axis_index("s")
    wid = s * _NC + c
    pltpu.sync_copy(zeros_hbm.at[pl.ds(0, _SLAB)],
                    acc.at[pl.ds(s * _SLAB, _SLAB)])

    @pl.when(s == 0)
    def _():
        pltpu.sync_copy(zeros_hbm.at[pl.ds(0, _TAIL)],
                        acc.at[pl.ds(_SLAB * _NS, _TAIL)])

    pltpu.sync_copy(ones_hbm, ones_v)
    pltpu.sync_copy(dst_hbm.at[pl.ds(wid * _NCHD, _NCHD)], didx)
    plsc.subcore_barrier()

    def step(j, carry):
        pltpu.async_copy(ones_v, acc.at[didx.at[2 * j]], dsem0, add=True)
        pltpu.async_copy(ones_v, acc.at[didx.at[2 * j + 1]], dsem1, add=True)
        pltpu.make_async_copy(ones_v, acc.at[didx.at[2 * j]], dsem0).wait()
        pltpu.make_async_copy(ones_v, acc.at[didx.at[2 * j + 1]], dsem1).wait()
        return carry

    lax.fori_loop(0, _NCHD // 2, step, 0)
    plsc.subcore_barrier()
    pltpu.sync_copy(acc.at[pl.ds(s * _SLAB, _SLAB)],
                    out_hbm.at[c, pl.ds(s * _SLAB, _SLAB)])

    @pl.when(s == 0)
    def _():
        pltpu.sync_copy(acc.at[pl.ds(_SLAB * _NS, _DTAIL)],
                        out_hbm.at[c, pl.ds(_SLAB * _NS, _DTAIL)])


_deg = pl.kernel(
    _deg_body,
    out_type=jax.ShapeDtypeStruct((_NC, _N, _D), jnp.float32),
    mesh=_mesh,
    scratch_types=[
        pltpu.VMEM((_NCHD, _CHD), jnp.int32),
        pltpu.VMEM((_CHD, _D), jnp.float32),
        pltpu.VMEM_SHARED((_NPAD, _D), jnp.float32),
        pltpu.SemaphoreType.DMA,
        pltpu.SemaphoreType.DMA,
    ],
)


def _agg_body(g_hbm, src_hbm, dst_hbm, zeros_hbm, out_hbm,
              sidx, didx, rows0, rows1, rows2, rows3,
              acc, semg0, semg1, semg2, semg3,
              sems0, sems1, sems2, sems3):
    c = lax.axis_index("c")
    s = lax.axis_index("s")
    rows = (rows0, rows1, rows2, rows3)
    semg = (semg0, semg1, semg2, semg3)
    sems = (sems0, sems1, sems2, sems3)
    pltpu.sync_copy(zeros_hbm.at[pl.ds(0, _SLAB)],
                    acc.at[pl.ds(s * _SLAB, _SLAB)])

    @pl.when(s == 0)
    def _():
        pltpu.sync_copy(zeros_hbm.at[pl.ds(0, _TAIL)],
                        acc.at[pl.ds(_SLAB * _NS, _TAIL)])

    plsc.subcore_barrier()

    base_chunk = jnp.where(c == _FAST, s * _CF, _NS * _CF + s * _CS)
    n_super = jnp.where(c == _FAST, _CF // _SG, _CS // _SG)

    def super_body(k, carry):
        cb = base_chunk + k * _SG
        pltpu.sync_copy(src_hbm.at[pl.ds(cb, _SG)], sidx)
        pltpu.sync_copy(dst_hbm.at[pl.ds(cb, _SG)], didx)

        def blk(j, carry2):
            for b in range(_PIPE):
                pltpu.async_copy(g_hbm.at[sidx.at[j * _PIPE + b]],
                                 rows[b], semg[b])
            for b in range(_PIPE):
                pltpu.make_async_copy(g_hbm.at[sidx.at[j * _PIPE + b]],
                                      rows[b], semg[b]).wait()
                pltpu.async_copy(rows[b], acc.at[didx.at[j * _PIPE + b]],
                                 sems[b], add=True)
            for b in range(_PIPE):
                pltpu.make_async_copy(rows[b], acc.at[didx.at[j * _PIPE + b]],
                                      sems[b]).wait()
            return carry2

        lax.fori_loop(0, _SG // _PIPE, blk, 0)
        return carry

    lax.fori_loop(0, n_super, super_body, 0)
    plsc.subcore_barrier()
    pltpu.sync_copy(acc.at[pl.ds(s * _SLAB, _SLAB)],
                    out_hbm.at[c, pl.ds(s * _SLAB, _SLAB)])

    @pl.when(s == 0)
    def _():
        pltpu.sync_copy(acc.at[pl.ds(_SLAB * _NS, _DTAIL)],
                        out_hbm.at[c, pl.ds(_SLAB * _NS, _DTAIL)])


_AGG_SCRATCH = (
    [
        pltpu.VMEM((_SG, _CH), jnp.int32),
        pltpu.VMEM((_SG, _CH), jnp.int32),
    ]
    + [pltpu.VMEM((_CH, _D), jnp.float32) for _ in range(_PIPE)]
    + [pltpu.VMEM_SHARED((_NPAD, _D), jnp.float32)]
    + [pltpu.SemaphoreType.DMA for _ in range(2 * _PIPE)]
)

_agg = pl.kernel(
    _agg_body,
    out_type=jax.ShapeDtypeStruct((_NC, _N, _D), jnp.float32),
    mesh=_mesh,
    scratch_types=_AGG_SCRATCH,
)


_BN = 1000


def _tc1_body(degp_ref, x_ref, w_ref, g_ref, dinv_ref):
    d = degp_ref[0][:, :1] + degp_ref[1][:, :1] + 1.0
    dinv = lax.rsqrt(jnp.maximum(d, 1e-12))
    h = jnp.dot(x_ref[...], w_ref[...], preferred_element_type=jnp.float32)
    g_ref[...] = dinv * h
    dinv_ref[...] = jnp.broadcast_to(dinv, dinv_ref.shape)


def _tc1(degp, x, w):
    return pl.pallas_call(
        _tc1_body,
        grid=(_N // _BN,),
        in_specs=[
            pl.BlockSpec((_NC, _BN, _D), lambda i: (0, i, 0)),
            pl.BlockSpec((_BN, _D), lambda i: (i, 0)),
            pl.BlockSpec((_D, _D), lambda i: (0, 0)),
        ],
        out_specs=[
            pl.BlockSpec((_BN, _D), lambda i: (i, 0)),
            pl.BlockSpec((_BN, 8), lambda i: (i, 0)),
        ],
        out_shape=[
            jax.ShapeDtypeStruct((_N, _D), jnp.float32),
            jax.ShapeDtypeStruct((_N, 8), jnp.float32),
        ],
    )(degp, x, w)


def _tc2_body(accp_ref, g1_ref, dinv_ref, b_ref, w_ref, g2_ref):
    sc = dinv_ref[:, :1]
    h = sc * (accp_ref[0] + accp_ref[1] + g1_ref[...]) + b_ref[...]
    h = jnp.maximum(h, 0.0)
    g2_ref[...] = sc * jnp.dot(h, w_ref[...],
                               preferred_element_type=jnp.float32)


def _tc2(accp, g1, dinv8, b, w):
    return pl.pallas_call(
        _tc2_body,
        grid=(_N // _BN,),
        in_specs=[
            pl.BlockSpec((_NC, _BN, _D), lambda i: (0, i, 0)),
            pl.BlockSpec((_BN, _D), lambda i: (i, 0)),
            pl.BlockSpec((_BN, 8), lambda i: (i, 0)),
            pl.BlockSpec((1, _D), lambda i: (0, 0)),
            pl.BlockSpec((_D, _D), lambda i: (0, 0)),
        ],
        out_specs=pl.BlockSpec((_BN, _D), lambda i: (i, 0)),
        out_shape=jax.ShapeDtypeStruct((_N, _D), jnp.float32),
    )(accp, g1, dinv8, b, w)


def _tc3_body(accp_ref, g2_ref, dinv_ref, b_ref, out_ref):
    sc = dinv_ref[:, :1]
    out_ref[...] = sc * (accp_ref[0] + accp_ref[1] + g2_ref[...]) + b_ref[...]


def _tc3(accp, g2, dinv8, b):
    return pl.pallas_call(
        _tc3_body,
        grid=(_N // _BN,),
        in_specs=[
            pl.BlockSpec((_NC, _BN, _D), lambda i: (0, i, 0)),
            pl.BlockSpec((_BN, _D), lambda i: (i, 0)),
            pl.BlockSpec((_BN, 8), lambda i: (i, 0)),
            pl.BlockSpec((1, _D), lambda i: (0, 0)),
        ],
        out_specs=pl.BlockSpec((_BN, _D), lambda i: (i, 0)),
        out_shape=jax.ShapeDtypeStruct((_N, _D), jnp.float32),
    )(accp, g2, dinv8, b)


def kernel(x, edge_index, W1, b1, W2, b2):
    src = edge_index[0]
    dst = edge_index[1]
    npad = _EPAD - _E
    src_pad = jnp.concatenate([src, jnp.zeros((npad,), jnp.int32)])
    dst_pad = jnp.concatenate([dst, jnp.full((npad,), _N, jnp.int32)])
    src64 = src_pad.reshape(_NCHA, _CH)
    dst64 = dst_pad.reshape(_NCHA, _CH)
    dst128 = dst_pad.reshape(_EPAD // _CHD, _CHD)
    zeros_d = jnp.zeros((_SLAB, _D), jnp.float32)
    ones_d = jnp.ones((_CHD, _D), jnp.float32)

    degp = _deg(dst128, ones_d, zeros_d)
    g1, dinv8 = _tc1(degp, x, W1)
    accp1 = _agg(g1, src64, dst64, zeros_d)
    g2 = _tc2(accp1, g1, dinv8, b1.reshape(1, _D), W2)
    accp2 = _agg(g2, src64, dst64, zeros_d)
    out = _tc3(accp2, g2, dinv8, b2.reshape(1, _D))
    return out


# speedup vs baseline: 1.0575x; 1.0575x over previous
import jax
import jax.numpy as jnp
from jax import lax
from jax.experimental import pallas as pl
from jax.experimental.pallas import tpu as pltpu
from jax.experimental.pallas import tpu_sc as plsc

_N = 10000
_E = 320000
_D = 128

_NC = 2
_NS = 16
_NW = _NC * _NS

_EPW = 10240
_EPAD = _EPW * _NW
_NPAD = _N + 8
_SLAB = 624
_TAIL = _NPAD - _SLAB * _NS
_DTAIL = _N - _SLAB * _NS

_CHD = 128
_NCHD = _EPW // _CHD

_CH = 64
_NCHA = _EPAD // _CH
_SG = 32
_PIPE = 4
_FAST = 0
_CF = 256
_CS = _NCHA // _NS - _CF

_mesh = plsc.VectorSubcoreMesh(core_axis_name="c", subcore_axis_name="s")


def _deg_body(dst_hbm, ones_hbm, zeros_hbm, out_hbm, didx, ones_v, acc,
              dsem0, dsem1):
    c = lax.axis_index("c")
    s = lax.axis_index("s")
    wid = s * _NC + c
    pltpu.sync_copy(zeros_hbm.at[pl.ds(0, _SLAB)],
                    acc.at[pl.ds(s * _SLAB, _SLAB)])

    @pl.when(s == 0)
    def _():
        pltpu.sync_copy(zeros_hbm.at[pl.ds(0, _TAIL)],
                        acc.at[pl.ds(_SLAB * _NS, _TAIL)])

    pltpu.sync_copy(ones_hbm, ones_v)
    pltpu.sync_copy(dst_hbm.at[pl.ds(wid * _NCHD, _NCHD)], didx)
    plsc.subcore_barrier()

    def step(j, carry):
        pltpu.async_copy(ones_v, acc.at[didx.at[2 * j]], dsem0, add=True)
        pltpu.async_copy(ones_v, acc.at[didx.at[2 * j + 1]], dsem1, add=True)
        pltpu.make_async_copy(ones_v, acc.at[didx.at[2 * j]], dsem0).wait()
        pltpu.make_async_copy(ones_v, acc.at[didx.at[2 * j + 1]], dsem1).wait()
        return carry

    lax.fori_loop(0, _NCHD // 2, step, 0)
    plsc.subcore_barrier()
    pltpu.sync_copy(acc.at[pl.ds(s * _SLAB, _SLAB)],
                    out_hbm.at[c, pl.ds(s * _SLAB, _SLAB)])

    @pl.when(s == 0)
    def _():
        pltpu.sync_copy(acc.at[pl.ds(_SLAB * _NS, _DTAIL)],
                        out_hbm.at[c, pl.ds(_SLAB * _NS, _DTAIL)])


_deg = pl.kernel(
    _deg_body,
    out_type=jax.ShapeDtypeStruct((_NC, _N, _D), jnp.float32),
    mesh=_mesh,
    scratch_types=[
        pltpu.VMEM((_NCHD, _CHD), jnp.int32),
        pltpu.VMEM((_CHD, _D), jnp.float32),
        pltpu.VMEM_SHARED((_NPAD, _D), jnp.float32),
        pltpu.SemaphoreType.DMA,
        pltpu.SemaphoreType.DMA,
    ],
)


def _agg_body(g_hbm, src_hbm, dst_hbm, zeros_hbm, out_hbm,
              sidx, didx, rows0, rows1, rows2, rows3,
              acc, semg0, semg1, semg2, semg3,
              sems0, sems1, sems2, sems3):
    c = lax.axis_index("c")
    s = lax.axis_index("s")
    rows = (rows0, rows1, rows2, rows3)
    semg = (semg0, semg1, semg2, semg3)
    sems = (sems0, sems1, sems2, sems3)
    pltpu.sync_copy(zeros_hbm.at[pl.ds(0, _SLAB)],
                    acc.at[pl.ds(s * _SLAB, _SLAB)])

    @pl.when(s == 0)
    def _():
        pltpu.sync_copy(zeros_hbm.at[pl.ds(0, _TAIL)],
                        acc.at[pl.ds(_SLAB * _NS, _TAIL)])

    plsc.subcore_barrier()

    base_chunk = jnp.where(c == _FAST, s * _CF, _NS * _CF + s * _CS)
    n_super = jnp.where(c == _FAST, _CF // _SG, _CS // _SG)

    def super_body(k, carry):
        cb = base_chunk + k * _SG
        pltpu.sync_copy(src_hbm.at[pl.ds(cb, _SG)], sidx)
        pltpu.sync_copy(dst_hbm.at[pl.ds(cb, _SG)], didx)

        def blk(j, carry2):
            for b in range(_PIPE):
                pltpu.async_copy(g_hbm.at[sidx.at[j * _PIPE + b]],
                                 rows[b], semg[b])
            for b in range(_PIPE):
                pltpu.make_async_copy(g_hbm.at[sidx.at[j * _PIPE + b]],
                                      rows[b], semg[b]).wait()
                pltpu.async_copy(rows[b], acc.at[didx.at[j * _PIPE + b]],
                                 sems[b], add=True)
            for b in range(_PIPE):
                pltpu.make_async_copy(rows[b], acc.at[didx.at[j * _PIPE + b]],
                                      sems[b]).wait()
            return carry2

        lax.fori_loop(0, _SG // _PIPE, blk, 0)
        return carry

    lax.fori_loop(0, n_super, super_body, 0)
    plsc.subcore_barrier()
    pltpu.sync_copy(acc.at[pl.ds(s * _SLAB, _SLAB)],
                    out_hbm.at[c, pl.ds(s * _SLAB, _SLAB)])

    @pl.when(s == 0)
    def _():
        pltpu.sync_copy(acc.at[pl.ds(_SLAB * _NS, _DTAIL)],
                        out_hbm.at[c, pl.ds(_SLAB * _NS, _DTAIL)])


_AGG_SCRATCH = (
    [
        pltpu.VMEM((_SG, _CH), jnp.int32),
        pltpu.VMEM((_SG, _CH), jnp.int32),
    ]
    + [pltpu.VMEM((_CH, _D), jnp.float32) for _ in range(_PIPE)]
    + [pltpu.VMEM_SHARED((_NPAD, _D), jnp.float32)]
    + [pltpu.SemaphoreType.DMA for _ in range(2 * _PIPE)]
)

_agg = pl.kernel(
    _agg_body,
    out_type=jax.ShapeDtypeStruct((_NC, _N, _D), jnp.float32),
    mesh=_mesh,
    scratch_types=_AGG_SCRATCH,
)


_BN = 1000


def _tc1_body(degp_ref, x_ref, w_ref, g_ref, dinv_ref):
    d = degp_ref[0][:, :1] + degp_ref[1][:, :1] + 1.0
    dinv = lax.rsqrt(jnp.maximum(d, 1e-12))
    h = jnp.dot(x_ref[...], w_ref[...], preferred_element_type=jnp.float32)
    g_ref[...] = dinv * h
    dinv_ref[...] = jnp.broadcast_to(dinv, dinv_ref.shape)


def _tc1(degp, x, w):
    return pl.pallas_call(
        _tc1_body,
        grid=(_N // _BN,),
        in_specs=[
            pl.BlockSpec((_NC, _BN, _D), lambda i: (0, i, 0)),
            pl.BlockSpec((_BN, _D), lambda i: (i, 0)),
            pl.BlockSpec((_D, _D), lambda i: (0, 0)),
        ],
        out_specs=[
            pl.BlockSpec((_BN, _D), lambda i: (i, 0)),
            pl.BlockSpec((_BN, 8), lambda i: (i, 0)),
        ],
        out_shape=[
            jax.ShapeDtypeStruct((_N, _D), jnp.float32),
            jax.ShapeDtypeStruct((_N, 8), jnp.float32),
        ],
    )(degp, x, w)


def _tc2_body(accp_ref, g1_ref, dinv_ref, b_ref, w_ref, g2_ref):
    sc = dinv_ref[:, :1]
    h = sc * (accp_ref[0] + accp_ref[1] + g1_ref[...]) + b_ref[...]
    h = jnp.maximum(h, 0.0)
    g2_ref[...] = sc * jnp.dot(h, w_ref[...],
                               preferred_element_type=jnp.float32)


def _tc2(accp, g1, dinv8, b, w):
    return pl.pallas_call(
        _tc2_body,
        grid=(_N // _BN,),
        in_specs=[
            pl.BlockSpec((_NC, _BN, _D), lambda i: (0, i, 0)),
            pl.BlockSpec((_BN, _D), lambda i: (i, 0)),
            pl.BlockSpec((_BN, 8), lambda i: (i, 0)),
            pl.BlockSpec((1, _D), lambda i: (0, 0)),
            pl.BlockSpec((_D, _D), lambda i: (0, 0)),
        ],
        out_specs=pl.BlockSpec((_BN, _D), lambda i: (i, 0)),
        out_shape=jax.ShapeDtypeStruct((_N, _D), jnp.float32),
    )(accp, g1, dinv8, b, w)


def _tc3_body(accp_ref, g2_ref, dinv_ref, b_ref, out_ref):
    sc = dinv_ref[:, :1]
    out_ref[...] = sc * (accp_ref[0] + accp_ref[1] + g2_ref[...]) + b_ref[...]


def _tc3(accp, g2, dinv8, b):
    return pl.pallas_call(
        _tc3_body,
        grid=(_N // _BN,),
        in_specs=[
            pl.BlockSpec((_NC, _BN, _D), lambda i: (0, i, 0)),
            pl.BlockSpec((_BN, _D), lambda i: (i, 0)),
            pl.BlockSpec((_BN, 8), lambda i: (i, 0)),
            pl.BlockSpec((1, _D), lambda i: (0, 0)),
        ],
        out_specs=pl.BlockSpec((_BN, _D), lambda i: (i, 0)),
        out_shape=jax.ShapeDtypeStruct((_N, _D), jnp.float32),
    )(accp, g2, dinv8, b)


def kernel(x, edge_index, W1, b1, W2, b2):
    src = edge_index[0]
    dst = edge_index[1]
    npad = _EPAD - _E
    src_pad = jnp.concatenate([src, jnp.zeros((npad,), jnp.int32)])
    dst_pad = jnp.concatenate([dst, jnp.full((npad,), _N, jnp.int32)])
    src64 = src_pad.reshape(_NCHA, _CH)
    dst64 = dst_pad.reshape(_NCHA, _CH)
    dst128 = dst_pad.reshape(_EPAD // _CHD, _CHD)
    zeros_d = jnp.zeros((_SLAB, _D), jnp.float32)
    ones_d = jnp.ones((_CHD, _D), jnp.float32)

    degp = _deg(dst128, ones_d, zeros_d)
    g1, dinv8 = _tc1(degp, x, W1)
    accp1 = _agg(g1, src64, dst64, zeros_d)
    g2 = _tc2(accp1, g1, dinv8, b1.reshape(1, _D), W2)
    accp2 = _agg(g2, src64, dst64, zeros_d)
    out = _tc3(accp2, g2, dinv8, b2.reshape(1, _D))
    return out
